# Initial kernel scaffold; baseline (speedup 1.0000x reference)
import jax
import jax.numpy as jnp
from jax.experimental import pallas as pl


def kernel(x, edge_index, adj, W1, b1, Wfc, bfc):
    raise NotImplementedError("write your pallas kernel here")


# speedup vs baseline: 27.2770x; 27.2770x over previous
import functools

import jax
import jax.numpy as jnp
from jax import lax
from jax.experimental import pallas as pl
from jax.experimental.pallas import tpu as pltpu
from jax.experimental.pallas import tpu_sc as plsc

NC = 2
NS = 16
NW = NC * NS
L = 16

_EDGE_CHUNK = 80


def _sc_degree(dst, n_pad):
    E = dst.shape[0]
    ept = E // NW
    mesh = plsc.VectorSubcoreMesh(core_axis_name="c", subcore_axis_name="s")

    @functools.partial(
        pl.kernel,
        out_type=jax.ShapeDtypeStruct((NW, n_pad), jnp.float32),
        mesh=mesh,
        scratch_types=[
            pltpu.VMEM((ept,), jnp.int32),
            pltpu.VMEM((n_pad,), jnp.float32),
        ],
        compiler_params=pltpu.CompilerParams(
            needs_layout_passes=False, use_tc_tiling_on_sc=False),
    )
    def deg_kernel(dst_hbm, out_hbm, idx_v, deg_v):
        c = lax.axis_index("c")
        s = lax.axis_index("s")
        wid = c * NS + s

        zeros16 = jnp.zeros((L,), jnp.float32)

        def zbody(i, carry):
            deg_v[pl.ds(i * L, L)] = zeros16
            return carry

        lax.fori_loop(0, n_pad // L, zbody, 0)

        pltpu.sync_copy(dst_hbm.at[pl.ds(wid * ept, ept)], idx_v)

        ones16 = jnp.ones((L,), jnp.float32)

        def body(i, carry):
            idx = idx_v[pl.ds(i * L, L)]
            plsc.addupdate_scatter(deg_v, [idx], ones16)
            return carry

        lax.fori_loop(0, ept // L, body, 0)
        pltpu.sync_copy(deg_v, out_hbm.at[wid])

    return deg_kernel(dst)


def _tc_scaled_matmul(x, W1, deg_parts):
    N, F = x.shape
    H = W1.shape[1]
    R = 512

    def body(x_ref, w_ref, deg_ref, out_ref):
        deg = jnp.sum(deg_ref[...], axis=0) + 1.0
        dis = lax.rsqrt(deg)
        h = jnp.dot(x_ref[...], w_ref[...], preferred_element_type=jnp.float32)
        out_ref[...] = h * dis[:, None]

    return pl.pallas_call(
        body,
        grid=(N // R,),
        in_specs=[
            pl.BlockSpec((R, F), lambda i: (i, 0)),
            pl.BlockSpec((F, H), lambda i: (0, 0)),
            pl.BlockSpec((NW, R), lambda i: (0, i)),
        ],
        out_specs=pl.BlockSpec((R, H), lambda i: (i, 0)),
        out_shape=jax.ShapeDtypeStruct((N, H), jnp.float32),
    )(x, W1, deg_parts)


def _sc_gather_scatter_add(g, src3, dst3, n_pad):
    _, n_chunk, C = src3.shape
    H = g.shape[1]
    rpt = n_pad // NS
    ZR = 80
    mesh = plsc.VectorSubcoreMesh(core_axis_name="c", subcore_axis_name="s")

    @functools.partial(
        pl.kernel,
        out_type=jax.ShapeDtypeStruct((NC, n_pad, H), jnp.float32),
        mesh=mesh,
        scratch_types=[
            pltpu.VMEM((n_chunk, C), jnp.int32),
            pltpu.VMEM((n_chunk, C), jnp.int32),
            pltpu.VMEM((C, H), jnp.float32),
            pltpu.VMEM((ZR, H), jnp.float32),
            pltpu.VMEM_SHARED((n_pad, H), jnp.float32),
            pltpu.SemaphoreType.DMA,
        ],
        compiler_params=pltpu.CompilerParams(
            needs_layout_passes=False, use_tc_tiling_on_sc=False),
    )
    def gs_kernel(g_hbm, src_hbm, dst_hbm, out_hbm,
                  src_v, dst_v, rows_v, z_v, acc_sh, sem):
        c = lax.axis_index("c")
        s = lax.axis_index("s")
        wid = c * NS + s

        zeros16 = jnp.zeros((L,), jnp.float32)

        def zrow(i, carry):
            def zcol(j, inner):
                z_v[i, pl.ds(j * L, L)] = zeros16
                return inner
            return lax.fori_loop(0, H // L, zcol, carry)

        lax.fori_loop(0, ZR, zrow, 0)

        r0 = s * rpt

        def zfill(k, carry):
            pltpu.sync_copy(z_v, acc_sh.at[pl.ds(r0 + k * ZR, ZR), :])
            return carry

        lax.fori_loop(0, rpt // ZR, zfill, 0)

        pltpu.sync_copy(src_hbm.at[wid], src_v)
        pltpu.sync_copy(dst_hbm.at[wid], dst_v)

        plsc.subcore_barrier()

        def body(j, carry):
            pltpu.async_copy(g_hbm.at[src_v.at[j]], rows_v, sem).wait()
            pltpu.sync_copy(rows_v, acc_sh.at[dst_v.at[j]], add=True)
            return carry

        lax.fori_loop(0, n_chunk, body, 0)

        plsc.subcore_barrier()

        pltpu.sync_copy(acc_sh.at[pl.ds(r0, rpt), :],
                        out_hbm.at[c, pl.ds(r0, rpt), :])

    return gs_kernel(g, src3, dst3)


def _tc_final(acc, g, deg_parts, b1, WfcT, bfc):
    N, H = g.shape
    R = 512

    def body(acc_ref, g_ref, deg_ref, b1_ref, w_ref, bfc_ref, out_ref):
        deg = jnp.sum(deg_ref[...], axis=0) + 1.0
        dis = lax.rsqrt(deg)
        t = (acc_ref[0] + acc_ref[1] + g_ref[...]) * dis[:, None] + b1_ref[...]
        out_ref[...] = (
            jnp.dot(t, w_ref[...], preferred_element_type=jnp.float32)
            + bfc_ref[...]
        )

    return pl.pallas_call(
        body,
        grid=(N // R,),
        in_specs=[
            pl.BlockSpec((NC, R, H), lambda i: (0, i, 0)),
            pl.BlockSpec((R, H), lambda i: (i, 0)),
            pl.BlockSpec((NW, R), lambda i: (0, i)),
            pl.BlockSpec((1, H), lambda i: (0, 0)),
            pl.BlockSpec((H, H), lambda i: (0, 0)),
            pl.BlockSpec((1, H), lambda i: (0, 0)),
        ],
        out_specs=pl.BlockSpec((R, H), lambda i: (i, 0)),
        out_shape=jax.ShapeDtypeStruct((N, H), jnp.float32),
    )(acc, g, deg_parts, b1, WfcT, bfc)


def kernel(x, edge_index, adj, W1, b1, Wfc, bfc):
    N, F = x.shape
    E = edge_index.shape[1]
    del adj

    src = edge_index[0].astype(jnp.int32)
    dst = edge_index[1].astype(jnp.int32)

    n_pad = ((N + NS * 32 - 1) // (NS * 32)) * (NS * 32)

    ept = E // NW
    n_chunk = ept // _EDGE_CHUNK
    src3 = src.reshape(NW, n_chunk, _EDGE_CHUNK)
    dst3 = dst.reshape(NW, n_chunk, _EDGE_CHUNK)

    x_pad = jnp.pad(x, ((0, n_pad - N), (0, 0)))

    deg_parts = _sc_degree(dst, n_pad)
    g = _tc_scaled_matmul(x_pad, W1, deg_parts)
    acc = _sc_gather_scatter_add(g, src3, dst3, n_pad)
    out = _tc_final(acc, g, deg_parts, b1.reshape(1, -1), Wfc.T,
                    bfc.reshape(1, -1))
    return out[:N]


# speedup vs baseline: 39.6263x; 1.4527x over previous
import functools

import jax
import jax.numpy as jnp
from jax import lax
from jax.experimental import pallas as pl
from jax.experimental.pallas import tpu as pltpu
from jax.experimental.pallas import tpu_sc as plsc

NC = 2
NS = 16
NW = NC * NS
L = 16

_EDGE_CHUNK = 100


def _sc_degree(dst, n_pad):
    E = dst.shape[0]
    ept = E // NW
    mesh = plsc.VectorSubcoreMesh(core_axis_name="c", subcore_axis_name="s")

    @functools.partial(
        pl.kernel,
        out_type=jax.ShapeDtypeStruct((NW, n_pad), jnp.float32),
        mesh=mesh,
        scratch_types=[
            pltpu.VMEM((ept,), jnp.int32),
            pltpu.VMEM((n_pad,), jnp.float32),
        ],
        compiler_params=pltpu.CompilerParams(
            needs_layout_passes=False, use_tc_tiling_on_sc=False),
    )
    def deg_kernel(dst_hbm, out_hbm, idx_v, deg_v):
        c = lax.axis_index("c")
        s = lax.axis_index("s")
        wid = c * NS + s

        zeros16 = jnp.zeros((L,), jnp.float32)

        def zbody(i, carry):
            deg_v[pl.ds(i * L, L)] = zeros16
            return carry

        lax.fori_loop(0, n_pad // L, zbody, 0)

        pltpu.sync_copy(dst_hbm.at[pl.ds(wid * ept, ept)], idx_v)

        ones16 = jnp.ones((L,), jnp.float32)

        def body(i, carry):
            idx = idx_v[pl.ds(i * L, L)]
            plsc.addupdate_scatter(deg_v, [idx], ones16)
            return carry

        lax.fori_loop(0, ept // L, body, 0)
        pltpu.sync_copy(deg_v, out_hbm.at[wid])

    return deg_kernel(dst)


def _tc_scaled_matmul(x, W1, deg_parts):
    N, F = x.shape
    H = W1.shape[1]
    R = 512

    def body(x_ref, w_ref, deg_ref, out_ref):
        deg = jnp.sum(deg_ref[...], axis=0) + 1.0
        dis = lax.rsqrt(deg)
        h = jnp.dot(x_ref[...], w_ref[...], preferred_element_type=jnp.float32)
        out_ref[...] = h * dis[:, None]

    return pl.pallas_call(
        body,
        grid=(N // R,),
        in_specs=[
            pl.BlockSpec((R, F), lambda i: (i, 0)),
            pl.BlockSpec((F, H), lambda i: (0, 0)),
            pl.BlockSpec((NW, R), lambda i: (0, i)),
        ],
        out_specs=pl.BlockSpec((R, H), lambda i: (i, 0)),
        out_shape=jax.ShapeDtypeStruct((N, H), jnp.float32),
    )(x, W1, deg_parts)


def _sc_gather_scatter_add(g, src3, dst3, n_pad):
    _, n_chunk, C = src3.shape
    H = g.shape[1]
    rpt = n_pad // NS
    ZR = 16
    mesh = plsc.VectorSubcoreMesh(core_axis_name="c", subcore_axis_name="s")

    @functools.partial(
        pl.kernel,
        out_type=jax.ShapeDtypeStruct((NC, n_pad, H), jnp.float32),
        mesh=mesh,
        scratch_types=[
            pltpu.VMEM((n_chunk, C), jnp.int32),
            pltpu.VMEM((n_chunk, C), jnp.int32),
            pltpu.VMEM((C, H), jnp.float32),
            pltpu.VMEM((C, H), jnp.float32),
            pltpu.VMEM((ZR, H), jnp.float32),
            pltpu.VMEM_SHARED((n_pad, H), jnp.float32),
            pltpu.SemaphoreType.DMA,
            pltpu.SemaphoreType.DMA,
        ],
        compiler_params=pltpu.CompilerParams(
            needs_layout_passes=False, use_tc_tiling_on_sc=False),
    )
    def gs_kernel(g_hbm, src_hbm, dst_hbm, out_hbm,
                  src_v, dst_v, rows0, rows1, z_v, acc_sh, sem0, sem1):
        c = lax.axis_index("c")
        s = lax.axis_index("s")
        wid = c * NS + s

        zeros16 = jnp.zeros((L,), jnp.float32)

        def zrow(i, carry):
            def zcol(j, inner):
                z_v[i, pl.ds(j * L, L)] = zeros16
                return inner
            return lax.fori_loop(0, H // L, zcol, carry)

        lax.fori_loop(0, ZR, zrow, 0)

        r0 = s * rpt

        def zfill(k, carry):
            pltpu.sync_copy(z_v, acc_sh.at[pl.ds(r0 + k * ZR, ZR), :])
            return carry

        lax.fori_loop(0, rpt // ZR, zfill, 0)

        pltpu.sync_copy(src_hbm.at[wid], src_v)
        pltpu.sync_copy(dst_hbm.at[wid], dst_v)

        plsc.subcore_barrier()

        pltpu.async_copy(g_hbm.at[src_v.at[0]], rows0, sem0)

        def body(b, carry):
            j0 = 2 * b
            j1 = j0 + 1
            pltpu.async_copy(g_hbm.at[src_v.at[j1]], rows1, sem1)
            pltpu.make_async_copy(g_hbm.at[src_v.at[j0]], rows0, sem0).wait()
            pltpu.sync_copy(rows0, acc_sh.at[dst_v.at[j0]], add=True)
            j2 = jnp.minimum(j0 + 2, n_chunk - 1)
            pltpu.async_copy(g_hbm.at[src_v.at[j2]], rows0, sem0)
            pltpu.make_async_copy(g_hbm.at[src_v.at[j1]], rows1, sem1).wait()
            pltpu.sync_copy(rows1, acc_sh.at[dst_v.at[j1]], add=True)
            return carry

        lax.fori_loop(0, n_chunk // 2, body, 0)
        pltpu.make_async_copy(g_hbm.at[src_v.at[0]], rows0, sem0).wait()

        plsc.subcore_barrier()

        pltpu.sync_copy(acc_sh.at[pl.ds(r0, rpt), :],
                        out_hbm.at[c, pl.ds(r0, rpt), :])

    return gs_kernel(g, src3, dst3)


def _tc_final(acc, g, deg_parts, b1, WfcT, bfc):
    N, H = g.shape
    R = 512

    def body(acc_ref, g_ref, deg_ref, b1_ref, w_ref, bfc_ref, out_ref):
        deg = jnp.sum(deg_ref[...], axis=0) + 1.0
        dis = lax.rsqrt(deg)
        t = (acc_ref[0] + acc_ref[1] + g_ref[...]) * dis[:, None] + b1_ref[...]
        out_ref[...] = (
            jnp.dot(t, w_ref[...], preferred_element_type=jnp.float32)
            + bfc_ref[...]
        )

    return pl.pallas_call(
        body,
        grid=(N // R,),
        in_specs=[
            pl.BlockSpec((NC, R, H), lambda i: (0, i, 0)),
            pl.BlockSpec((R, H), lambda i: (i, 0)),
            pl.BlockSpec((NW, R), lambda i: (0, i)),
            pl.BlockSpec((1, H), lambda i: (0, 0)),
            pl.BlockSpec((H, H), lambda i: (0, 0)),
            pl.BlockSpec((1, H), lambda i: (0, 0)),
        ],
        out_specs=pl.BlockSpec((R, H), lambda i: (i, 0)),
        out_shape=jax.ShapeDtypeStruct((N, H), jnp.float32),
    )(acc, g, deg_parts, b1, WfcT, bfc)


def kernel(x, edge_index, adj, W1, b1, Wfc, bfc):
    N, F = x.shape
    E = edge_index.shape[1]
    del adj

    src = edge_index[0].astype(jnp.int32)
    dst = edge_index[1].astype(jnp.int32)

    n_pad = ((N + NS * 32 - 1) // (NS * 32)) * (NS * 32)

    ept = E // NW
    n_chunk = ept // _EDGE_CHUNK
    src3 = src.reshape(NW, n_chunk, _EDGE_CHUNK)
    dst3 = dst.reshape(NW, n_chunk, _EDGE_CHUNK)

    x_pad = jnp.pad(x, ((0, n_pad - N), (0, 0)))

    deg_parts = _sc_degree(dst, n_pad)
    g = _tc_scaled_matmul(x_pad, W1, deg_parts)
    acc = _sc_gather_scatter_add(g, src3, dst3, n_pad)
    out = _tc_final(acc, g, deg_parts, b1.reshape(1, -1), Wfc.T,
                    bfc.reshape(1, -1))
    return out[:N]
